# baseline (speedup 1.0000x reference)
import functools

import jax
import jax.numpy as jnp
from jax import lax
from jax.experimental import pallas as pl
from jax.experimental.pallas import tpu as pltpu
from jax.experimental.pallas import tpu_sc as plsc

NUM_CORES = 2
NUM_SUBCORES = 16
NW = NUM_CORES * NUM_SUBCORES

BATCH = 16384
HIST = 50
DIM = 64
TOTAL = BATCH * HIST
PER_W = TOTAL // NW
CHUNK = 512
NCHUNK = PER_W // CHUNK
NBUF = 2
NOUT = NCHUNK // NBUF

BBLK = 128

_mesh = plsc.VectorSubcoreMesh(
    core_axis_name="c", subcore_axis_name="s",
    num_cores=NUM_CORES, num_subcores=NUM_SUBCORES,
)


@functools.partial(
    pl.kernel,
    out_type=jax.ShapeDtypeStruct((TOTAL, 2 * DIM), jnp.float32),
    mesh=_mesh,
    scratch_types=[
        pltpu.VMEM((PER_W,), jnp.int32),
        *[pltpu.VMEM((CHUNK, DIM), jnp.float32) for _ in range(NBUF)],
        *[pltpu.SemaphoreType.DMA for _ in range(NBUF)],
        *[pltpu.SemaphoreType.DMA for _ in range(NBUF)],
    ],
    compiler_params=pltpu.CompilerParams(use_tc_tiling_on_sc=False),
)
def _gather_sc(idx_hbm, table_hbm, out_hbm, idx_v, *scratch):
    bufs = scratch[:NBUF]
    gsem = scratch[NBUF:2 * NBUF]
    osem = scratch[2 * NBUF:]

    wid = lax.axis_index("s") * NUM_CORES + lax.axis_index("c")
    row0 = wid * PER_W
    pltpu.sync_copy(idx_hbm.at[wid], idx_v)

    def fire_gather(j, buf, sem):
        pltpu.async_copy(
            table_hbm.at[idx_v.at[pl.ds(j * CHUNK, CHUNK)]], buf, sem)

    def wait_gather(buf, sem):
        pltpu.make_async_copy(
            table_hbm.at[pl.ds(0, CHUNK)], buf, sem).wait()

    def fire_writeback(j, buf, sem):
        pltpu.async_copy(
            buf,
            out_hbm.at[pl.ds(row0 + j * CHUNK, CHUNK), pl.ds(0, DIM)], sem)

    def wait_writeback(buf, sem):
        pltpu.make_async_copy(
            buf, out_hbm.at[pl.ds(0, CHUNK), pl.ds(0, DIM)], sem).wait()

    for b in range(NBUF):
        fire_gather(b, bufs[b], gsem[b])

    def body(t, carry):
        j0 = t * NBUF
        for b in range(NBUF):
            j = j0 + b
            wait_gather(bufs[b], gsem[b])
            fire_writeback(j, bufs[b], osem[b])

            @pl.when(j + NBUF < NCHUNK)
            def _():
                wait_writeback(bufs[b], osem[b])
                fire_gather(j + NBUF, bufs[b], gsem[b])
        return carry

    lax.fori_loop(0, NOUT, body, 0)
    for b in range(NBUF):
        wait_writeback(bufs[b], osem[b])


def _fmt_tc(rows_ref, out_ref):
    x = rows_ref[...]
    out_ref[...] = x[:, :DIM].reshape(BBLK, HIST, DIM)


_format = pl.pallas_call(
    _fmt_tc,
    grid=(BATCH // BBLK,),
    in_specs=[pl.BlockSpec((BBLK * HIST, 2 * DIM), lambda i: (i, 0))],
    out_specs=pl.BlockSpec((BBLK, HIST, DIM), lambda i: (i, 0, 0)),
    out_shape=jax.ShapeDtypeStruct((BATCH, HIST, DIM), jnp.float32),
)


def kernel(input_ids, weight):
    idx = input_ids.reshape(NW, PER_W).astype(jnp.int32)
    rows = _gather_sc(idx, weight)
    return _format(rows)


# speedup vs baseline: 1.0072x; 1.0072x over previous
import functools

import jax
import jax.numpy as jnp
from jax import lax
from jax.experimental import pallas as pl
from jax.experimental.pallas import tpu as pltpu
from jax.experimental.pallas import tpu_sc as plsc

NUM_CORES = 2
NUM_SUBCORES = 16
NW = NUM_CORES * NUM_SUBCORES

BATCH = 16384
HIST = 50
DIM = 64
TOTAL = BATCH * HIST
PER_W = TOTAL // NW
CHUNK = 512
NCHUNK = PER_W // CHUNK
NBUF = 2
NOUT = NCHUNK // NBUF

BBLK = 128

_mesh = plsc.VectorSubcoreMesh(
    core_axis_name="c", subcore_axis_name="s",
    num_cores=NUM_CORES, num_subcores=NUM_SUBCORES,
)


@functools.partial(
    pl.kernel,
    out_type=jax.ShapeDtypeStruct((TOTAL, 2 * DIM), jnp.float32),
    mesh=_mesh,
    scratch_types=[
        pltpu.VMEM((PER_W,), jnp.int32),
        *[pltpu.VMEM((CHUNK, DIM), jnp.float32) for _ in range(NBUF)],
        *[pltpu.SemaphoreType.DMA for _ in range(NBUF)],
        *[pltpu.SemaphoreType.DMA for _ in range(NBUF)],
    ],
    compiler_params=pltpu.CompilerParams(use_tc_tiling_on_sc=False),
)
def _gather_sc(idx_hbm, table_hbm, out_hbm, idx_v, *scratch):
    bufs = scratch[:NBUF]
    gsem = scratch[NBUF:2 * NBUF]
    osem = scratch[2 * NBUF:]

    wid = lax.axis_index("s") * NUM_CORES + lax.axis_index("c")
    row0 = wid * PER_W
    pltpu.sync_copy(idx_hbm.at[wid], idx_v)

    def fire_gather(j, buf, sem):
        pltpu.async_copy(
            table_hbm.at[idx_v.at[pl.ds(j * CHUNK, CHUNK)]], buf, sem)

    def wait_gather(buf, sem):
        pltpu.make_async_copy(
            table_hbm.at[pl.ds(0, CHUNK)], buf, sem).wait()

    def fire_writeback(j, buf, sem):
        pltpu.async_copy(
            buf,
            out_hbm.at[pl.ds(row0 + j * CHUNK, CHUNK), pl.ds(0, DIM)], sem)

    def wait_writeback(buf, sem):
        pltpu.make_async_copy(
            buf, out_hbm.at[pl.ds(0, CHUNK), pl.ds(0, DIM)], sem).wait()

    for b in range(NBUF):
        fire_gather(b, bufs[b], gsem[b])

    def body(t, carry):
        j0 = t * NBUF
        for b in range(NBUF):
            j = j0 + b
            wait_gather(bufs[b], gsem[b])
            fire_writeback(j, bufs[b], osem[b])

            @pl.when(j + NBUF < NCHUNK)
            def _():
                wait_writeback(bufs[b], osem[b])
                fire_gather(j + NBUF, bufs[b], gsem[b])
        return carry

    lax.fori_loop(0, NOUT, body, 0)
    for b in range(NBUF):
        wait_writeback(bufs[b], osem[b])


def _fmt_tc(rows_ref, out_ref):
    x = rows_ref[...]
    out_ref[...] = x[:, :DIM].reshape(BBLK, HIST, DIM)


_format = pl.pallas_call(
    _fmt_tc,
    grid=(BATCH // BBLK,),
    in_specs=[pl.BlockSpec((BBLK * HIST, 2 * DIM), lambda i: (i, 0))],
    out_specs=pl.BlockSpec((BBLK, HIST, DIM), lambda i: (i, 0, 0)),
    out_shape=jax.ShapeDtypeStruct((BATCH, HIST, DIM), jnp.float32),
)


def kernel(input_ids, weight):
    idx = input_ids.reshape(NW, PER_W).astype(jnp.int32)
    rows = _gather_sc(idx, weight)
    return rows[:, :DIM].reshape(BATCH, HIST, DIM)


# speedup vs baseline: 1.0826x; 1.0749x over previous
import functools

import jax
import jax.numpy as jnp
from jax import lax
from jax.experimental import pallas as pl
from jax.experimental.pallas import tpu as pltpu
from jax.experimental.pallas import tpu_sc as plsc

NUM_CORES = 2
NUM_SUBCORES = 16
NW = NUM_CORES * NUM_SUBCORES

BATCH = 16384
HIST = 50
DIM = 64
TOTAL = BATCH * HIST
PER_W = TOTAL // NW
B_PER_W = BATCH // NW
BCHUNK = 8
CHUNK = BCHUNK * HIST
NCHUNK = B_PER_W // BCHUNK
NBUF = 2
NOUT = NCHUNK // NBUF

_mesh = plsc.VectorSubcoreMesh(
    core_axis_name="c", subcore_axis_name="s",
    num_cores=NUM_CORES, num_subcores=NUM_SUBCORES,
)


@functools.partial(
    pl.kernel,
    out_type=jax.ShapeDtypeStruct((TOTAL, DIM), jnp.float32),
    mesh=_mesh,
    scratch_types=[
        pltpu.VMEM((B_PER_W, HIST), jnp.int32),
        *[pltpu.VMEM((CHUNK, DIM), jnp.float32) for _ in range(NBUF)],
        *[pltpu.SemaphoreType.DMA for _ in range(NBUF)],
        *[pltpu.SemaphoreType.DMA for _ in range(NBUF)],
    ],
    compiler_params=pltpu.CompilerParams(use_tc_tiling_on_sc=False),
)
def _gather_sc(idx_hbm, table_hbm, out_hbm, idx_v, *scratch):
    bufs = scratch[:NBUF]
    gsem = scratch[NBUF:2 * NBUF]
    osem = scratch[2 * NBUF:]

    wid = lax.axis_index("s") * NUM_CORES + lax.axis_index("c")
    row0 = wid * PER_W
    pltpu.sync_copy(idx_hbm.at[pl.ds(wid * B_PER_W, B_PER_W)], idx_v)

    def fire_gather(j, buf, sem):
        for k in range(BCHUNK):
            pltpu.async_copy(
                table_hbm.at[idx_v.at[j * BCHUNK + k]],
                buf.at[pl.ds(k * HIST, HIST)], sem)

    def wait_gather(buf, sem):
        for k in range(BCHUNK):
            pltpu.make_async_copy(
                table_hbm.at[pl.ds(0, HIST)],
                buf.at[pl.ds(0, HIST)], sem).wait()

    def fire_writeback(j, buf, sem):
        pltpu.async_copy(buf, out_hbm.at[pl.ds(row0 + j * CHUNK, CHUNK)], sem)

    def wait_writeback(buf, sem):
        pltpu.make_async_copy(
            buf, out_hbm.at[pl.ds(0, CHUNK)], sem).wait()

    for b in range(NBUF):
        fire_gather(b, bufs[b], gsem[b])

    def body(t, carry):
        j0 = t * NBUF
        for b in range(NBUF):
            j = j0 + b
            wait_gather(bufs[b], gsem[b])
            fire_writeback(j, bufs[b], osem[b])

            @pl.when(j + NBUF < NCHUNK)
            def _():
                wait_writeback(bufs[b], osem[b])
                fire_gather(j + NBUF, bufs[b], gsem[b])
        return carry

    lax.fori_loop(0, NOUT, body, 0)
    for b in range(NBUF):
        wait_writeback(bufs[b], osem[b])


def kernel(input_ids, weight):
    rows = _gather_sc(input_ids.astype(jnp.int32), weight)
    return rows.reshape(BATCH, HIST, DIM)
